# Initial kernel scaffold; baseline (speedup 1.0000x reference)
import jax
import jax.numpy as jnp
from jax.experimental import pallas as pl


def kernel(features, edge_index, edge_weight, W):
    raise NotImplementedError("write your pallas kernel here")


# speedup vs baseline: 4.1410x; 4.1410x over previous
import functools

import jax
import jax.numpy as jnp
from jax import lax
from jax.experimental import pallas as pl
from jax.experimental.pallas import tpu as pltpu
from jax.experimental.pallas import tpu_sc as plsc

N = 10000
E = 320000
D = 128

NC = 2
NS = 16
NW = NC * NS

E_PER_TILE = E // NW
CHUNK = 80
NCHUNKS = E_PER_TILE // CHUNK

N_PAD = 10240
ROWS_PER_TILE = N_PAD // NS
ZROWS = 128


_GATHER_DNUMS = lax.GatherDimensionNumbers(
    offset_dims=(), collapsed_slice_dims=(0,), start_index_map=(0,))


def _bcast_lane(vec16, lane):
    idx = jnp.full((16, 1), lane, jnp.int32)
    return lax.gather(vec16, idx, _GATHER_DNUMS, (1,),
                      mode=lax.GatherScatterMode.PROMISE_IN_BOUNDS)


def _mm_body(x_ref, w_ref, o_ref):
    o_ref[...] = jnp.dot(x_ref[...], w_ref[...], preferred_element_type=jnp.float32)


def _fin_body(a_ref, b_ref, o_ref):
    o_ref[...] = jnp.maximum(a_ref[0] + b_ref[0], 0.0)


def _sc_body(support, src, dst, w, out, src_v, dst_v, w_v, rows_v, zbuf, acc, sem):
    c = lax.axis_index("c")
    s = lax.axis_index("s")
    wid = s * NC + c

    def zrow(i, _):
        for j in range(D // 16):
            zbuf[i, pl.ds(16 * j, 16)] = jnp.zeros((16,), jnp.float32)
        return _
    lax.fori_loop(0, ZROWS, zrow, 0)
    row_base = s * ROWS_PER_TILE
    for k in range(ROWS_PER_TILE // ZROWS):
        pltpu.sync_copy(zbuf, acc.at[pl.ds(row_base + k * ZROWS, ZROWS)])
    plsc.subcore_barrier()

    def chunk_body(g, _):
        base = wid * E_PER_TILE + g * CHUNK
        pltpu.sync_copy(src.at[pl.ds(base, CHUNK)], src_v)
        pltpu.sync_copy(dst.at[pl.ds(base, CHUNK)], dst_v)
        pltpu.sync_copy(w.at[pl.ds(base, CHUNK)], w_v)
        pltpu.async_copy(support.at[src_v], rows_v, sem).wait()

        def group_body(g2, _):
            e0 = g2 * 16
            wv16 = w_v[pl.ds(e0, 16)]
            for l in range(16):
                wb = _bcast_lane(wv16, l)
                e = e0 + l
                for j in range(D // 16):
                    rows_v[e, pl.ds(16 * j, 16)] = rows_v[e, pl.ds(16 * j, 16)] * wb
            return _
        lax.fori_loop(0, CHUNK // 16, group_body, 0)

        pltpu.sync_copy(rows_v, acc.at[dst_v], add=True)
        return _
    lax.fori_loop(0, NCHUNKS, chunk_body, 0)
    plsc.subcore_barrier()

    for k in range(ROWS_PER_TILE // ZROWS):
        r0 = row_base + k * ZROWS
        pltpu.sync_copy(acc.at[pl.ds(r0, ZROWS)], out.at[c, pl.ds(r0, ZROWS)])


@functools.partial(
    pl.kernel,
    out_type=jax.ShapeDtypeStruct((NC, N_PAD, D), jnp.float32),
    mesh=plsc.VectorSubcoreMesh(core_axis_name="c", subcore_axis_name="s"),
    scratch_types=[
        pltpu.VMEM((CHUNK,), jnp.int32),
        pltpu.VMEM((CHUNK,), jnp.int32),
        pltpu.VMEM((CHUNK,), jnp.float32),
        pltpu.VMEM((CHUNK, D), jnp.float32),
        pltpu.VMEM((ZROWS, D), jnp.float32),
        pltpu.VMEM_SHARED((N_PAD, D), jnp.float32),
        pltpu.SemaphoreType.DMA,
    ],
)
def _sc_aggregate(support, src, dst, w, out, *scratch):
    _sc_body(support, src, dst, w, out, *scratch)


_BLK = 1000


def kernel(features, edge_index, edge_weight, W):
    support = pl.pallas_call(
        _mm_body,
        grid=(N // _BLK,),
        in_specs=[
            pl.BlockSpec((_BLK, D), lambda i: (i, 0)),
            pl.BlockSpec((D, D), lambda i: (0, 0)),
        ],
        out_specs=pl.BlockSpec((_BLK, D), lambda i: (i, 0)),
        out_shape=jax.ShapeDtypeStruct((N, D), jnp.float32),
    )(features, W)

    parts = _sc_aggregate(support, edge_index[0], edge_index[1], edge_weight)

    out = pl.pallas_call(
        _fin_body,
        grid=(N // _BLK,),
        in_specs=[
            pl.BlockSpec((1, _BLK, D), lambda i: (0, i, 0)),
            pl.BlockSpec((1, _BLK, D), lambda i: (1, i, 0)),
        ],
        out_specs=pl.BlockSpec((_BLK, D), lambda i: (i, 0)),
        out_shape=jax.ShapeDtypeStruct((N, D), jnp.float32),
    )(parts, parts)
    return out
